# baseline (speedup 1.0000x reference)
import functools
import math

import jax
import jax.numpy as jnp
from jax import lax
from jax.experimental import pallas as pl
from jax.experimental.pallas import tpu as pltpu
from jax.experimental.pallas import tpu_sc as plsc

B = 1
S = 2048
H = 1024
NH = 16
HD = H // NH
E = 8
K = 2
F = 2816

TS = 256
TB = 128
NST = S // TS
NP = K * S + E * TB
NT = NP // TB


def _attn_router_body(x_ref, g1_ref, b1_ref, wq_ref, bq_ref, wk_ref, bk_ref,
                      wv_ref, bv_ref, g2_ref, b2_ref, wr_ref, br_ref,
                      hidden_ref, x2_ref, gates_ref):
    x = x_ref[...]
    mu = jnp.mean(x, axis=-1, keepdims=True)
    var = jnp.mean((x - mu) ** 2, axis=-1, keepdims=True)
    xn = (x - mu) / jnp.sqrt(var + 1e-5) * g1_ref[...] + b1_ref[...]

    xn16 = xn.astype(jnp.bfloat16)
    q = jnp.dot(xn16, wq_ref[...], preferred_element_type=jnp.float32) + bq_ref[...]
    k = jnp.dot(xn16, wk_ref[...], preferred_element_type=jnp.float32) + bk_ref[...]
    v = jnp.dot(xn16, wv_ref[...], preferred_element_type=jnp.float32) + bv_ref[...]

    jj = lax.broadcasted_iota(jnp.int32, (H, NH), 0)
    mm = lax.broadcasted_iota(jnp.int32, (H, NH), 1)
    R = (jj // HD == mm).astype(jnp.float32)
    scale = 1.0 / math.sqrt(HD)
    qf = q.astype(jnp.bfloat16).astype(jnp.float32)
    kf = k.astype(jnp.bfloat16).astype(jnp.float32)
    vf = v.astype(jnp.bfloat16).astype(jnp.float32)
    ctx_parts = []
    for n in range(NH):
        qn = qf[:, n * HD:(n + 1) * HD]
        qt = jnp.concatenate([qn] * NH, axis=1)
        s = jnp.dot(qt * kf, R, preferred_element_type=jnp.float32,
                    precision=lax.Precision.HIGHEST) * scale
        s = s - jnp.max(s, axis=-1, keepdims=True)
        es = jnp.exp(s)
        probs = es / jnp.sum(es, axis=-1, keepdims=True)
        probs = probs.astype(jnp.bfloat16).astype(jnp.float32)
        cn = jnp.zeros((TS, HD), jnp.float32)
        for m in range(NH):
            cn = cn + probs[:, m:m + 1] * vf[:, m * HD:(m + 1) * HD]
        ctx_parts.append(cn)
    ctx = jnp.concatenate(ctx_parts, axis=1)

    hidden = x + ctx
    hidden_ref[...] = hidden

    mu2 = jnp.mean(hidden, axis=-1, keepdims=True)
    var2 = jnp.mean((hidden - mu2) ** 2, axis=-1, keepdims=True)
    x2 = (hidden - mu2) / jnp.sqrt(var2 + 1e-5) * g2_ref[...] + b2_ref[...]
    x2_16 = x2.astype(jnp.bfloat16)
    x2_ref[...] = x2_16

    logits = jnp.dot(x2_16, wr_ref[...],
                     preferred_element_type=jnp.float32) + br_ref[...]
    lm = jnp.max(logits, axis=-1, keepdims=True)
    el = jnp.exp(logits - lm)
    rw = el / jnp.sum(el, axis=-1, keepdims=True)

    iota_e = lax.broadcasted_iota(jnp.int32, (TS, E), 1)
    m1 = jnp.max(rw, axis=-1, keepdims=True)
    i1 = jnp.min(jnp.where(rw == m1, iota_e, E), axis=-1, keepdims=True)
    oh1 = iota_e == i1
    masked = jnp.where(oh1, -jnp.inf, rw)
    m2 = jnp.max(masked, axis=-1, keepdims=True)
    i2 = jnp.min(jnp.where(masked == m2, iota_e, E), axis=-1, keepdims=True)
    oh2 = iota_e == i2
    w1 = jnp.sum(jnp.where(oh1, rw, 0.0), axis=-1, keepdims=True)
    w2 = jnp.sum(jnp.where(oh2, rw, 0.0), axis=-1, keepdims=True)
    gates = (oh1 * w1 + oh2 * w2) / (w1 + w2)
    gates_ref[...] = gates.astype(jnp.float32)


def _make_sc_gather(d, b_rows):
    info = plsc.get_sparse_core_info()
    nw = info.num_cores * info.num_subcores
    b_per_w = b_rows // nw
    chunk = b_per_w // 2
    assert b_per_w % 2 == 0 and chunk % 8 == 0
    mesh = plsc.VectorSubcoreMesh(core_axis_name="c", subcore_axis_name="s")

    @functools.partial(
        pl.kernel, mesh=mesh,
        out_type=jax.ShapeDtypeStruct((b_rows, d), jnp.int32),
        scratch_types=[
            pltpu.VMEM((chunk,), jnp.int32),
            pltpu.VMEM((chunk, d), jnp.int32),
            pltpu.SemaphoreType.DMA,
        ],
    )
    def g(table_hbm, idx_hbm, out_hbm, idx_v, rows_v, sem):
        wid = lax.axis_index("s") * info.num_cores + lax.axis_index("c")
        base = wid * b_per_w
        for c in range(b_per_w // chunk):
            o = base + c * chunk
            pltpu.sync_copy(idx_hbm.at[pl.ds(o, chunk)], idx_v)
            pltpu.async_copy(table_hbm.at[idx_v], rows_v, sem).wait()
            pltpu.sync_copy(rows_v, out_hbm.at[pl.ds(o, chunk)])

    return g


def _moe_grouped_body(eot_ref, xs_ref, w1_ref, b1_ref, w2_ref, b2_ref, ys_ref):
    xb = xs_ref[...]
    h = jnp.dot(xb, w1_ref[0], preferred_element_type=jnp.float32) + b1_ref[0]
    a = 0.5 * h * (1.0 + lax.erf(h / math.sqrt(2.0)))
    y = jnp.dot(a.astype(jnp.bfloat16), w2_ref[0],
                preferred_element_type=jnp.float32) + b2_ref[0]
    ys_ref[...] = y.astype(jnp.bfloat16)


def _combine_body(hidden_ref, gates_ref, y1_ref, y2_ref, out_ref):
    g = gates_ref[...]
    w1 = jnp.max(g, axis=-1, keepdims=True)
    w2 = jnp.sum(g, axis=-1, keepdims=True) - w1
    out_ref[...] = (hidden_ref[...] + w1 * y1_ref[...].astype(jnp.float32)
                    + w2 * y2_ref[...].astype(jnp.float32))


def kernel(hidden_states, ln1_g, ln1_b, Wq, bq, Wk, bk, Wv, bv, Wl, bl,
           ln2_g, ln2_b, Wr, br, W1, b1, W2, b2):
    x = hidden_states.reshape(S, H)
    bf = jnp.bfloat16
    Wq16, Wk16, Wv16, Wr16 = (w.astype(bf) for w in (Wq, Wk, Wv, Wr))
    W1_16, W2_16 = W1.astype(bf), W2.astype(bf)

    row = lambda a: a.reshape(1, -1)

    hidden, x2, gates = pl.pallas_call(
        _attn_router_body,
        grid=(NST,),
        in_specs=[
            pl.BlockSpec((TS, H), lambda s: (s, 0)),
            pl.BlockSpec((1, H), lambda s: (0, 0)),
            pl.BlockSpec((1, H), lambda s: (0, 0)),
            pl.BlockSpec((H, H), lambda s: (0, 0)),
            pl.BlockSpec((1, H), lambda s: (0, 0)),
            pl.BlockSpec((H, H), lambda s: (0, 0)),
            pl.BlockSpec((1, H), lambda s: (0, 0)),
            pl.BlockSpec((H, H), lambda s: (0, 0)),
            pl.BlockSpec((1, H), lambda s: (0, 0)),
            pl.BlockSpec((1, H), lambda s: (0, 0)),
            pl.BlockSpec((1, H), lambda s: (0, 0)),
            pl.BlockSpec((H, E), lambda s: (0, 0)),
            pl.BlockSpec((1, E), lambda s: (0, 0)),
        ],
        out_specs=[
            pl.BlockSpec((TS, H), lambda s: (s, 0)),
            pl.BlockSpec((TS, H), lambda s: (s, 0)),
            pl.BlockSpec((TS, E), lambda s: (s, 0)),
        ],
        out_shape=[
            jax.ShapeDtypeStruct((S, H), jnp.float32),
            jax.ShapeDtypeStruct((S, H), jnp.bfloat16),
            jax.ShapeDtypeStruct((S, E), jnp.float32),
        ],
    )(x, row(ln1_g), row(ln1_b), Wq16, row(bq), Wk16, row(bk), Wv16, row(bv),
      row(ln2_g), row(ln2_b), Wr16, row(br))

    iota_e = jnp.arange(E, dtype=jnp.int32)[None, :]
    i1 = jnp.argmax(gates, axis=-1).astype(jnp.int32)
    masked = jnp.where(iota_e == i1[:, None], -1.0, gates)
    i2 = jnp.argmax(masked, axis=-1).astype(jnp.int32)
    e_all = jnp.concatenate([i1, i2])
    oh = (e_all[:, None] == iota_e).astype(jnp.int32)
    cum = jnp.cumsum(oh, axis=0)
    rank = jnp.take_along_axis(cum, e_all[:, None], axis=1)[:, 0] - 1
    counts = cum[-1]
    padded = ((counts + TB - 1) // TB) * TB
    ends = jnp.cumsum(padded)
    off = ends - padded
    dest = off[e_all] + rank
    src_tok = jnp.concatenate([jnp.arange(S, dtype=jnp.int32)] * 2)
    src_idx = jnp.zeros((NP,), jnp.int32).at[dest].set(src_tok)
    pos12 = dest
    t_starts = jnp.arange(NT, dtype=jnp.int32) * TB
    eot = jnp.minimum(
        jnp.sum((t_starts[:, None] >= ends[None, :]).astype(jnp.int32), axis=1),
        E - 1).astype(jnp.int32)

    to_i32 = lambda a: lax.bitcast_convert_type(
        a.reshape(a.shape[0], a.shape[1] // 2, 2), jnp.int32)
    to_bf16 = lambda a: lax.bitcast_convert_type(
        a, jnp.bfloat16).reshape(a.shape[0], -1)
    xs = to_bf16(_make_sc_gather(H // 2, NP)(to_i32(x2), src_idx))

    ys = pl.pallas_call(
        _moe_grouped_body,
        grid_spec=pltpu.PrefetchScalarGridSpec(
            num_scalar_prefetch=1,
            grid=(NT,),
            in_specs=[
                pl.BlockSpec((TB, H), lambda t, eot_ref: (t, 0)),
                pl.BlockSpec((1, H, F), lambda t, eot_ref: (eot_ref[t], 0, 0)),
                pl.BlockSpec((1, 1, F), lambda t, eot_ref: (eot_ref[t], 0, 0)),
                pl.BlockSpec((1, F, H), lambda t, eot_ref: (eot_ref[t], 0, 0)),
                pl.BlockSpec((1, 1, H), lambda t, eot_ref: (eot_ref[t], 0, 0)),
            ],
            out_specs=pl.BlockSpec((TB, H), lambda t, eot_ref: (t, 0)),
        ),
        out_shape=jax.ShapeDtypeStruct((NP, H), jnp.bfloat16),
    )(eot, xs, W1_16, b1.reshape(E, 1, F), W2_16, b2.reshape(E, 1, H))

    y12 = to_bf16(_make_sc_gather(H // 2, K * S)(to_i32(ys), pos12))

    out = pl.pallas_call(
        _combine_body,
        grid=(NST,),
        in_specs=[
            pl.BlockSpec((TS, H), lambda s: (s, 0)),
            pl.BlockSpec((TS, E), lambda s: (s, 0)),
            pl.BlockSpec((TS, H), lambda s: (s, 0)),
            pl.BlockSpec((TS, H), lambda s: (s + NST, 0)),
        ],
        out_specs=pl.BlockSpec((TS, H), lambda s: (s, 0)),
        out_shape=jax.ShapeDtypeStruct((S, H), jnp.float32),
    )(hidden, gates, y12, y12)

    return out.reshape(B, S, H)


# speedup vs baseline: 1.6571x; 1.6571x over previous
import functools
import math

import jax
import jax.numpy as jnp
from jax import lax
from jax.experimental import pallas as pl
from jax.experimental.pallas import tpu as pltpu
from jax.experimental.pallas import tpu_sc as plsc

B = 1
S = 2048
H = 1024
NH = 16
HD = H // NH
E = 8
K = 2
F = 2816

TS = 256
TB = 128
NST = S // TS
NP = K * S + E * TB
NT = NP // TB


def _attn_router_body(x_ref, g1_ref, b1_ref, wq_ref, bq_ref, wk_ref, bk_ref,
                      wv_ref, bv_ref, g2_ref, b2_ref, wr_ref, br_ref,
                      hidden_ref, x2_ref, gates_ref):
    x = x_ref[...]
    mu = jnp.mean(x, axis=-1, keepdims=True)
    var = jnp.mean((x - mu) ** 2, axis=-1, keepdims=True)
    xn = (x - mu) / jnp.sqrt(var + 1e-5) * g1_ref[...] + b1_ref[...]

    xn16 = xn.astype(jnp.bfloat16)
    q = jnp.dot(xn16, wq_ref[...], preferred_element_type=jnp.float32) + bq_ref[...]
    k = jnp.dot(xn16, wk_ref[...], preferred_element_type=jnp.float32) + bk_ref[...]
    v = jnp.dot(xn16, wv_ref[...], preferred_element_type=jnp.float32) + bv_ref[...]

    jj = lax.broadcasted_iota(jnp.int32, (H, NH), 0)
    mm = lax.broadcasted_iota(jnp.int32, (H, NH), 1)
    R = (jj // HD == mm).astype(jnp.float32)
    scale = 1.0 / math.sqrt(HD)
    qf = q.astype(jnp.bfloat16).astype(jnp.float32)
    kf = k.astype(jnp.bfloat16).astype(jnp.float32)
    vf = v.astype(jnp.bfloat16).astype(jnp.float32)
    ctx_parts = []
    for n in range(NH):
        qn = qf[:, n * HD:(n + 1) * HD]
        qt = jnp.concatenate([qn] * NH, axis=1)
        s = jnp.dot(qt * kf, R, preferred_element_type=jnp.float32,
                    precision=lax.Precision.HIGHEST) * scale
        s = s - jnp.max(s, axis=-1, keepdims=True)
        es = jnp.exp(s)
        probs = es / jnp.sum(es, axis=-1, keepdims=True)
        probs = probs.astype(jnp.bfloat16).astype(jnp.float32)
        cn = jnp.zeros((TS, HD), jnp.float32)
        for m in range(NH):
            cn = cn + probs[:, m:m + 1] * vf[:, m * HD:(m + 1) * HD]
        ctx_parts.append(cn)
    ctx = jnp.concatenate(ctx_parts, axis=1)

    hidden = x + ctx
    hidden_ref[...] = hidden

    mu2 = jnp.mean(hidden, axis=-1, keepdims=True)
    var2 = jnp.mean((hidden - mu2) ** 2, axis=-1, keepdims=True)
    x2 = (hidden - mu2) / jnp.sqrt(var2 + 1e-5) * g2_ref[...] + b2_ref[...]
    x2_16 = x2.astype(jnp.bfloat16)
    x2_ref[...] = x2

    logits = jnp.dot(x2_16, wr_ref[...],
                     preferred_element_type=jnp.float32) + br_ref[...]
    lm = jnp.max(logits, axis=-1, keepdims=True)
    el = jnp.exp(logits - lm)
    rw = el / jnp.sum(el, axis=-1, keepdims=True)

    iota_e = lax.broadcasted_iota(jnp.int32, (TS, E), 1)
    m1 = jnp.max(rw, axis=-1, keepdims=True)
    i1 = jnp.min(jnp.where(rw == m1, iota_e, E), axis=-1, keepdims=True)
    oh1 = iota_e == i1
    masked = jnp.where(oh1, -jnp.inf, rw)
    m2 = jnp.max(masked, axis=-1, keepdims=True)
    i2 = jnp.min(jnp.where(masked == m2, iota_e, E), axis=-1, keepdims=True)
    oh2 = iota_e == i2
    w1 = jnp.sum(jnp.where(oh1, rw, 0.0), axis=-1, keepdims=True)
    w2 = jnp.sum(jnp.where(oh2, rw, 0.0), axis=-1, keepdims=True)
    gates = (oh1 * w1 + oh2 * w2) / (w1 + w2)
    gates_ref[...] = gates.astype(jnp.float32)


def _make_sc_gather(d, b_rows, n_chunks=4):
    info = plsc.get_sparse_core_info()
    nw = info.num_cores * info.num_subcores
    b_per_w = b_rows // nw
    chunk = b_per_w // n_chunks
    assert b_per_w % n_chunks == 0 and chunk % 8 == 0
    mesh = plsc.VectorSubcoreMesh(core_axis_name="c", subcore_axis_name="s")

    @functools.partial(
        pl.kernel, mesh=mesh,
        out_type=jax.ShapeDtypeStruct((b_rows, d), jnp.float32),
        scratch_types=[
            pltpu.VMEM((b_per_w,), jnp.int32),
            pltpu.VMEM((chunk, d), jnp.float32),
            pltpu.VMEM((chunk, d), jnp.float32),
            pltpu.SemaphoreType.DMA,
            pltpu.SemaphoreType.DMA,
            pltpu.SemaphoreType.DMA,
            pltpu.SemaphoreType.DMA,
        ],
    )
    def g(table_hbm, idx_hbm, out_hbm, idx_v, rows_a, rows_b, g_a, g_b, o_a, o_b):
        wid = lax.axis_index("s") * info.num_cores + lax.axis_index("c")
        base = wid * b_per_w
        pltpu.sync_copy(idx_hbm.at[pl.ds(base, b_per_w)], idx_v)
        rows = (rows_a, rows_b)
        gsem = (g_a, g_b)
        osem = (o_a, o_b)
        gh = [None, None]
        oh = [None, None]
        gh[0] = pltpu.async_copy(
            table_hbm.at[idx_v.at[pl.ds(0, chunk)]], rows[0], gsem[0])
        for c in range(n_chunks):
            b = c % 2
            nb = (c + 1) % 2
            if c + 1 < n_chunks:
                if oh[nb] is not None:
                    oh[nb].wait()
                gh[nb] = pltpu.async_copy(
                    table_hbm.at[idx_v.at[pl.ds((c + 1) * chunk, chunk)]],
                    rows[nb], gsem[nb])
            gh[b].wait()
            oh[b] = pltpu.async_copy(
                rows[b], out_hbm.at[pl.ds(base + c * chunk, chunk)], osem[b])
        oh[(n_chunks - 2) % 2].wait()
        oh[(n_chunks - 1) % 2].wait()

    return g


def _moe_grouped_body(eot_ref, xs_ref, w1_ref, b1_ref, w2_ref, b2_ref, ys_ref):
    xb = xs_ref[...].astype(jnp.bfloat16)
    h = jnp.dot(xb, w1_ref[0], preferred_element_type=jnp.float32) + b1_ref[0]
    a = 0.5 * h * (1.0 + lax.erf(h / math.sqrt(2.0)))
    ys_ref[...] = jnp.dot(a.astype(jnp.bfloat16), w2_ref[0],
                          preferred_element_type=jnp.float32) + b2_ref[0]


def _combine_body(hidden_ref, gates_ref, y1_ref, y2_ref, out_ref):
    g = gates_ref[...]
    w1 = jnp.max(g, axis=-1, keepdims=True)
    w2 = jnp.sum(g, axis=-1, keepdims=True) - w1
    out_ref[...] = (hidden_ref[...] + w1 * y1_ref[...].astype(jnp.float32)
                    + w2 * y2_ref[...].astype(jnp.float32))


def kernel(hidden_states, ln1_g, ln1_b, Wq, bq, Wk, bk, Wv, bv, Wl, bl,
           ln2_g, ln2_b, Wr, br, W1, b1, W2, b2):
    x = hidden_states.reshape(S, H)
    bf = jnp.bfloat16
    Wq16, Wk16, Wv16, Wr16 = (w.astype(bf) for w in (Wq, Wk, Wv, Wr))
    W1_16, W2_16 = W1.astype(bf), W2.astype(bf)

    row = lambda a: a.reshape(1, -1)

    hidden, x2, gates = pl.pallas_call(
        _attn_router_body,
        grid=(NST,),
        in_specs=[
            pl.BlockSpec((TS, H), lambda s: (s, 0)),
            pl.BlockSpec((1, H), lambda s: (0, 0)),
            pl.BlockSpec((1, H), lambda s: (0, 0)),
            pl.BlockSpec((H, H), lambda s: (0, 0)),
            pl.BlockSpec((1, H), lambda s: (0, 0)),
            pl.BlockSpec((H, H), lambda s: (0, 0)),
            pl.BlockSpec((1, H), lambda s: (0, 0)),
            pl.BlockSpec((H, H), lambda s: (0, 0)),
            pl.BlockSpec((1, H), lambda s: (0, 0)),
            pl.BlockSpec((1, H), lambda s: (0, 0)),
            pl.BlockSpec((1, H), lambda s: (0, 0)),
            pl.BlockSpec((H, E), lambda s: (0, 0)),
            pl.BlockSpec((1, E), lambda s: (0, 0)),
        ],
        out_specs=[
            pl.BlockSpec((TS, H), lambda s: (s, 0)),
            pl.BlockSpec((TS, H), lambda s: (s, 0)),
            pl.BlockSpec((TS, E), lambda s: (s, 0)),
        ],
        out_shape=[
            jax.ShapeDtypeStruct((S, H), jnp.float32),
            jax.ShapeDtypeStruct((S, H), jnp.float32),
            jax.ShapeDtypeStruct((S, E), jnp.float32),
        ],
    )(x, row(ln1_g), row(ln1_b), Wq16, row(bq), Wk16, row(bk), Wv16, row(bv),
      row(ln2_g), row(ln2_b), Wr16, row(br))

    iota_e = jnp.arange(E, dtype=jnp.int32)[None, :]
    i1 = jnp.argmax(gates, axis=-1).astype(jnp.int32)
    masked = jnp.where(iota_e == i1[:, None], -1.0, gates)
    i2 = jnp.argmax(masked, axis=-1).astype(jnp.int32)
    e_all = jnp.concatenate([i1, i2])
    oh = (e_all[:, None] == iota_e).astype(jnp.int32)
    cum = jnp.cumsum(oh, axis=0)
    rank = jnp.take_along_axis(cum, e_all[:, None], axis=1)[:, 0] - 1
    counts = cum[-1]
    padded = ((counts + TB - 1) // TB) * TB
    ends = jnp.cumsum(padded)
    off = ends - padded
    dest = off[e_all] + rank
    src_tok = jnp.concatenate([jnp.arange(S, dtype=jnp.int32)] * 2)
    src_idx = jnp.zeros((NP,), jnp.int32).at[dest].set(src_tok)
    pos12 = dest
    t_starts = jnp.arange(NT, dtype=jnp.int32) * TB
    eot = jnp.minimum(
        jnp.sum((t_starts[:, None] >= ends[None, :]).astype(jnp.int32), axis=1),
        E - 1).astype(jnp.int32)

    xs = _make_sc_gather(H, NP)(x2, src_idx)

    ys = pl.pallas_call(
        _moe_grouped_body,
        grid_spec=pltpu.PrefetchScalarGridSpec(
            num_scalar_prefetch=1,
            grid=(NT,),
            in_specs=[
                pl.BlockSpec((TB, H), lambda t, eot_ref: (t, 0)),
                pl.BlockSpec((1, H, F), lambda t, eot_ref: (eot_ref[t], 0, 0)),
                pl.BlockSpec((1, 1, F), lambda t, eot_ref: (eot_ref[t], 0, 0)),
                pl.BlockSpec((1, F, H), lambda t, eot_ref: (eot_ref[t], 0, 0)),
                pl.BlockSpec((1, 1, H), lambda t, eot_ref: (eot_ref[t], 0, 0)),
            ],
            out_specs=pl.BlockSpec((TB, H), lambda t, eot_ref: (t, 0)),
        ),
        out_shape=jax.ShapeDtypeStruct((NP, H), jnp.float32),
    )(eot, xs, W1_16, b1.reshape(E, 1, F), W2_16, b2.reshape(E, 1, H))

    y12 = _make_sc_gather(H, K * S)(ys, pos12)

    out = pl.pallas_call(
        _combine_body,
        grid=(NST,),
        in_specs=[
            pl.BlockSpec((TS, H), lambda s: (s, 0)),
            pl.BlockSpec((TS, E), lambda s: (s, 0)),
            pl.BlockSpec((TS, H), lambda s: (s, 0)),
            pl.BlockSpec((TS, H), lambda s: (s + NST, 0)),
        ],
        out_specs=pl.BlockSpec((TS, H), lambda s: (s, 0)),
        out_shape=jax.ShapeDtypeStruct((S, H), jnp.float32),
    )(hidden, gates, y12, y12)

    return out.reshape(B, S, H)


# speedup vs baseline: 1.7286x; 1.0431x over previous
import functools
import math

import jax
import jax.numpy as jnp
from jax import lax
from jax.experimental import pallas as pl
from jax.experimental.pallas import tpu as pltpu
from jax.experimental.pallas import tpu_sc as plsc

B = 1
S = 2048
H = 1024
NH = 16
HD = H // NH
E = 8
K = 2
F = 2816

TS = 256
TB = 128
NST = S // TS
NP = K * S + E * TB
NT = NP // TB


def _attn_router_body(x_ref, g1_ref, b1_ref, wq_ref, bq_ref, wk_ref, bk_ref,
                      wv_ref, bv_ref, g2_ref, b2_ref, wr_ref, br_ref,
                      hidden_ref, x2_ref, gates_ref):
    x = x_ref[...]
    mu = jnp.mean(x, axis=-1, keepdims=True)
    var = jnp.mean((x - mu) ** 2, axis=-1, keepdims=True)
    xn = (x - mu) / jnp.sqrt(var + 1e-5) * g1_ref[...] + b1_ref[...]

    xn16 = xn.astype(jnp.bfloat16)
    q = jnp.dot(xn16, wq_ref[...], preferred_element_type=jnp.float32) + bq_ref[...]
    k = jnp.dot(xn16, wk_ref[...], preferred_element_type=jnp.float32) + bk_ref[...]
    v = jnp.dot(xn16, wv_ref[...], preferred_element_type=jnp.float32) + bv_ref[...]

    jj = lax.broadcasted_iota(jnp.int32, (H, NH), 0)
    mm = lax.broadcasted_iota(jnp.int32, (H, NH), 1)
    R = (jj // HD == mm).astype(jnp.float32)
    scale = 1.0 / math.sqrt(HD)
    qf = q.astype(jnp.bfloat16).astype(jnp.float32)
    kf = k.astype(jnp.bfloat16).astype(jnp.float32)
    vf = v.astype(jnp.bfloat16).astype(jnp.float32)
    ctx_parts = []
    for n in range(NH):
        qn = qf[:, n * HD:(n + 1) * HD]
        qt = jnp.concatenate([qn] * NH, axis=1)
        s = jnp.dot(qt * kf, R, preferred_element_type=jnp.float32,
                    precision=lax.Precision.HIGHEST) * scale
        s = s - jnp.max(s, axis=-1, keepdims=True)
        es = jnp.exp(s)
        probs = es / jnp.sum(es, axis=-1, keepdims=True)
        probs = probs.astype(jnp.bfloat16).astype(jnp.float32)
        cn = jnp.zeros((TS, HD), jnp.float32)
        for m in range(NH):
            cn = cn + probs[:, m:m + 1] * vf[:, m * HD:(m + 1) * HD]
        ctx_parts.append(cn)
    ctx = jnp.concatenate(ctx_parts, axis=1)

    hidden = x + ctx
    hidden_ref[...] = hidden

    mu2 = jnp.mean(hidden, axis=-1, keepdims=True)
    var2 = jnp.mean((hidden - mu2) ** 2, axis=-1, keepdims=True)
    x2 = (hidden - mu2) / jnp.sqrt(var2 + 1e-5) * g2_ref[...] + b2_ref[...]
    x2_16 = x2.astype(jnp.bfloat16)
    x2_ref[...] = _pack16(x2_16)

    logits = jnp.dot(x2_16, wr_ref[...],
                     preferred_element_type=jnp.float32) + br_ref[...]
    lm = jnp.max(logits, axis=-1, keepdims=True)
    el = jnp.exp(logits - lm)
    rw = el / jnp.sum(el, axis=-1, keepdims=True)

    iota_e = lax.broadcasted_iota(jnp.int32, (TS, E), 1)
    m1 = jnp.max(rw, axis=-1, keepdims=True)
    i1 = jnp.min(jnp.where(rw == m1, iota_e, E), axis=-1, keepdims=True)
    oh1 = iota_e == i1
    masked = jnp.where(oh1, -jnp.inf, rw)
    m2 = jnp.max(masked, axis=-1, keepdims=True)
    i2 = jnp.min(jnp.where(masked == m2, iota_e, E), axis=-1, keepdims=True)
    oh2 = iota_e == i2
    w1 = jnp.sum(jnp.where(oh1, rw, 0.0), axis=-1, keepdims=True)
    w2 = jnp.sum(jnp.where(oh2, rw, 0.0), axis=-1, keepdims=True)
    gates = (oh1 * w1 + oh2 * w2) / (w1 + w2)
    gates_ref[...] = gates.astype(jnp.float32)


HP = H // 2


def _pack16(a16):
    ai = lax.bitcast_convert_type(a16, jnp.int16)
    lo = ai[:, :HP].astype(jnp.int32) & 0xFFFF
    hi = ai[:, HP:].astype(jnp.int32) << 16
    return lo | hi


def _unpack16(p):
    lo = lax.bitcast_convert_type(p << 16, jnp.float32)
    hi = lax.bitcast_convert_type(p & jnp.int32(-65536), jnp.float32)
    return jnp.concatenate([lo, hi], axis=1)


def _make_sc_gather(d, b_rows, dtype, n_chunks=4):
    info = plsc.get_sparse_core_info()
    nw = info.num_cores * info.num_subcores
    b_per_w = b_rows // nw
    chunk = b_per_w // n_chunks
    assert b_per_w % n_chunks == 0 and chunk % 8 == 0
    mesh = plsc.VectorSubcoreMesh(core_axis_name="c", subcore_axis_name="s")

    @functools.partial(
        pl.kernel, mesh=mesh,
        out_type=jax.ShapeDtypeStruct((b_rows, d), dtype),
        scratch_types=[
            pltpu.VMEM((b_per_w,), jnp.int32),
            pltpu.VMEM((chunk, d), dtype),
            pltpu.VMEM((chunk, d), dtype),
            pltpu.SemaphoreType.DMA,
            pltpu.SemaphoreType.DMA,
            pltpu.SemaphoreType.DMA,
            pltpu.SemaphoreType.DMA,
        ],
    )
    def g(table_hbm, idx_hbm, out_hbm, idx_v, rows_a, rows_b, g_a, g_b, o_a, o_b):
        wid = lax.axis_index("s") * info.num_cores + lax.axis_index("c")
        base = wid * b_per_w
        pltpu.sync_copy(idx_hbm.at[pl.ds(base, b_per_w)], idx_v)
        rows = (rows_a, rows_b)
        gsem = (g_a, g_b)
        osem = (o_a, o_b)
        gh = [None, None]
        oh = [None, None]
        gh[0] = pltpu.async_copy(
            table_hbm.at[idx_v.at[pl.ds(0, chunk)]], rows[0], gsem[0])
        for c in range(n_chunks):
            b = c % 2
            nb = (c + 1) % 2
            if c + 1 < n_chunks:
                if oh[nb] is not None:
                    oh[nb].wait()
                gh[nb] = pltpu.async_copy(
                    table_hbm.at[idx_v.at[pl.ds((c + 1) * chunk, chunk)]],
                    rows[nb], gsem[nb])
            gh[b].wait()
            oh[b] = pltpu.async_copy(
                rows[b], out_hbm.at[pl.ds(base + c * chunk, chunk)], osem[b])
        oh[(n_chunks - 2) % 2].wait()
        oh[(n_chunks - 1) % 2].wait()

    return g


def _moe_grouped_body(eot_ref, xs_ref, w1_ref, b1_ref, w2_ref, b2_ref, ys_ref):
    xb = _unpack16(xs_ref[...]).astype(jnp.bfloat16)
    h = jnp.dot(xb, w1_ref[0], preferred_element_type=jnp.float32) + b1_ref[0]
    a = 0.5 * h * (1.0 + lax.erf(h / math.sqrt(2.0)))
    y = jnp.dot(a.astype(jnp.bfloat16), w2_ref[0],
                preferred_element_type=jnp.float32) + b2_ref[0]
    ys_ref[...] = _pack16(y.astype(jnp.bfloat16))


def _combine_body(hidden_ref, gates_ref, y1_ref, y2_ref, out_ref):
    g = gates_ref[...]
    w1 = jnp.max(g, axis=-1, keepdims=True)
    w2 = jnp.sum(g, axis=-1, keepdims=True) - w1
    y1 = _unpack16(y1_ref[...])
    y2 = _unpack16(y2_ref[...])
    out_ref[...] = hidden_ref[...] + w1 * y1 + w2 * y2


def kernel(hidden_states, ln1_g, ln1_b, Wq, bq, Wk, bk, Wv, bv, Wl, bl,
           ln2_g, ln2_b, Wr, br, W1, b1, W2, b2):
    x = hidden_states.reshape(S, H)
    bf = jnp.bfloat16
    Wq16, Wk16, Wv16, Wr16 = (w.astype(bf) for w in (Wq, Wk, Wv, Wr))
    W1_16, W2_16 = W1.astype(bf), W2.astype(bf)

    row = lambda a: a.reshape(1, -1)

    hidden, x2, gates = pl.pallas_call(
        _attn_router_body,
        grid=(NST,),
        in_specs=[
            pl.BlockSpec((TS, H), lambda s: (s, 0)),
            pl.BlockSpec((1, H), lambda s: (0, 0)),
            pl.BlockSpec((1, H), lambda s: (0, 0)),
            pl.BlockSpec((H, H), lambda s: (0, 0)),
            pl.BlockSpec((1, H), lambda s: (0, 0)),
            pl.BlockSpec((H, H), lambda s: (0, 0)),
            pl.BlockSpec((1, H), lambda s: (0, 0)),
            pl.BlockSpec((H, H), lambda s: (0, 0)),
            pl.BlockSpec((1, H), lambda s: (0, 0)),
            pl.BlockSpec((1, H), lambda s: (0, 0)),
            pl.BlockSpec((1, H), lambda s: (0, 0)),
            pl.BlockSpec((H, E), lambda s: (0, 0)),
            pl.BlockSpec((1, E), lambda s: (0, 0)),
        ],
        out_specs=[
            pl.BlockSpec((TS, H), lambda s: (s, 0)),
            pl.BlockSpec((TS, HP), lambda s: (s, 0)),
            pl.BlockSpec((TS, E), lambda s: (s, 0)),
        ],
        out_shape=[
            jax.ShapeDtypeStruct((S, H), jnp.float32),
            jax.ShapeDtypeStruct((S, HP), jnp.int32),
            jax.ShapeDtypeStruct((S, E), jnp.float32),
        ],
    )(x, row(ln1_g), row(ln1_b), Wq16, row(bq), Wk16, row(bk), Wv16, row(bv),
      row(ln2_g), row(ln2_b), Wr16, row(br))

    iota_e = jnp.arange(E, dtype=jnp.int32)[None, :]
    i1 = jnp.argmax(gates, axis=-1).astype(jnp.int32)
    masked = jnp.where(iota_e == i1[:, None], -1.0, gates)
    i2 = jnp.argmax(masked, axis=-1).astype(jnp.int32)
    e_all = jnp.concatenate([i1, i2])
    oh = (e_all[:, None] == iota_e).astype(jnp.int32)
    cum = jnp.cumsum(oh, axis=0)
    rank = jnp.take_along_axis(cum, e_all[:, None], axis=1)[:, 0] - 1
    counts = cum[-1]
    padded = ((counts + TB - 1) // TB) * TB
    ends = jnp.cumsum(padded)
    off = ends - padded
    dest = off[e_all] + rank
    src_tok = jnp.concatenate([jnp.arange(S, dtype=jnp.int32)] * 2)
    src_idx = jnp.zeros((NP,), jnp.int32).at[dest].set(src_tok)
    pos12 = dest
    t_starts = jnp.arange(NT, dtype=jnp.int32) * TB
    eot = jnp.minimum(
        jnp.sum((t_starts[:, None] >= ends[None, :]).astype(jnp.int32), axis=1),
        E - 1).astype(jnp.int32)

    xs = _make_sc_gather(HP, NP, jnp.int32)(x2, src_idx)

    ys = pl.pallas_call(
        _moe_grouped_body,
        grid_spec=pltpu.PrefetchScalarGridSpec(
            num_scalar_prefetch=1,
            grid=(NT,),
            in_specs=[
                pl.BlockSpec((TB, HP), lambda t, eot_ref: (t, 0)),
                pl.BlockSpec((1, H, F), lambda t, eot_ref: (eot_ref[t], 0, 0)),
                pl.BlockSpec((1, 1, F), lambda t, eot_ref: (eot_ref[t], 0, 0)),
                pl.BlockSpec((1, F, H), lambda t, eot_ref: (eot_ref[t], 0, 0)),
                pl.BlockSpec((1, 1, H), lambda t, eot_ref: (eot_ref[t], 0, 0)),
            ],
            out_specs=pl.BlockSpec((TB, HP), lambda t, eot_ref: (t, 0)),
        ),
        out_shape=jax.ShapeDtypeStruct((NP, HP), jnp.int32),
    )(eot, xs, W1_16, b1.reshape(E, 1, F), W2_16, b2.reshape(E, 1, H))

    y12 = _make_sc_gather(HP, K * S, jnp.int32)(ys, pos12)

    out = pl.pallas_call(
        _combine_body,
        grid=(NST,),
        in_specs=[
            pl.BlockSpec((TS, H), lambda s: (s, 0)),
            pl.BlockSpec((TS, E), lambda s: (s, 0)),
            pl.BlockSpec((TS, HP), lambda s: (s, 0)),
            pl.BlockSpec((TS, HP), lambda s: (s + NST, 0)),
        ],
        out_specs=pl.BlockSpec((TS, H), lambda s: (s, 0)),
        out_shape=jax.ShapeDtypeStruct((S, H), jnp.float32),
    )(hidden, gates, y12, y12)

    return out.reshape(B, S, H)


# speedup vs baseline: 1.7346x; 1.0035x over previous
import functools
import math

import jax
import jax.numpy as jnp
from jax import lax
from jax.experimental import pallas as pl
from jax.experimental.pallas import tpu as pltpu
from jax.experimental.pallas import tpu_sc as plsc

B = 1
S = 2048
H = 1024
NH = 16
HD = H // NH
E = 8
K = 2
F = 2816

TS = 256
TB = 128
NST = S // TS
NP = K * S + E * TB
NT = NP // TB


def _attn_router_body(x_ref, g1_ref, b1_ref, wq_ref, bq_ref, wk_ref, bk_ref,
                      wv_ref, bv_ref, g2_ref, b2_ref, wr_ref, br_ref,
                      hidden_ref, x2_ref, gates_ref):
    x = x_ref[...]
    mu = jnp.mean(x, axis=-1, keepdims=True)
    var = jnp.mean((x - mu) ** 2, axis=-1, keepdims=True)
    xn = (x - mu) / jnp.sqrt(var + 1e-5) * g1_ref[...] + b1_ref[...]

    xn16 = xn.astype(jnp.bfloat16)
    q = jnp.dot(xn16, wq_ref[...], preferred_element_type=jnp.float32) + bq_ref[...]
    k = jnp.dot(xn16, wk_ref[...], preferred_element_type=jnp.float32) + bk_ref[...]
    v = jnp.dot(xn16, wv_ref[...], preferred_element_type=jnp.float32) + bv_ref[...]

    jj = lax.broadcasted_iota(jnp.int32, (H, NH), 0)
    mm = lax.broadcasted_iota(jnp.int32, (H, NH), 1)
    R = (jj // HD == mm).astype(jnp.float32)
    scale = 1.0 / math.sqrt(HD)
    qf = q.astype(jnp.bfloat16).astype(jnp.float32)
    kf = k.astype(jnp.bfloat16).astype(jnp.float32)
    vf = v.astype(jnp.bfloat16).astype(jnp.float32)
    ctx_parts = []
    for n in range(NH):
        qn = qf[:, n * HD:(n + 1) * HD]
        qt = jnp.concatenate([qn] * NH, axis=1)
        s = jnp.dot(qt * kf, R, preferred_element_type=jnp.float32,
                    precision=lax.Precision.HIGHEST) * scale
        s = s - jnp.max(s, axis=-1, keepdims=True)
        es = jnp.exp(s)
        probs = es / jnp.sum(es, axis=-1, keepdims=True)
        probs = probs.astype(jnp.bfloat16).astype(jnp.float32)
        cn = jnp.zeros((TS, HD), jnp.float32)
        for m in range(NH):
            cn = cn + probs[:, m:m + 1] * vf[:, m * HD:(m + 1) * HD]
        ctx_parts.append(cn)
    ctx = jnp.concatenate(ctx_parts, axis=1)

    hidden = x + ctx
    hidden_ref[...] = hidden

    mu2 = jnp.mean(hidden, axis=-1, keepdims=True)
    var2 = jnp.mean((hidden - mu2) ** 2, axis=-1, keepdims=True)
    x2 = (hidden - mu2) / jnp.sqrt(var2 + 1e-5) * g2_ref[...] + b2_ref[...]
    x2_16 = x2.astype(jnp.bfloat16)
    x2_ref[...] = _pack16(x2_16)

    logits = jnp.dot(x2_16, wr_ref[...],
                     preferred_element_type=jnp.float32) + br_ref[...]
    lm = jnp.max(logits, axis=-1, keepdims=True)
    el = jnp.exp(logits - lm)
    rw = el / jnp.sum(el, axis=-1, keepdims=True)

    iota_e = lax.broadcasted_iota(jnp.int32, (TS, E), 1)
    m1 = jnp.max(rw, axis=-1, keepdims=True)
    i1 = jnp.min(jnp.where(rw == m1, iota_e, E), axis=-1, keepdims=True)
    oh1 = iota_e == i1
    masked = jnp.where(oh1, -jnp.inf, rw)
    m2 = jnp.max(masked, axis=-1, keepdims=True)
    i2 = jnp.min(jnp.where(masked == m2, iota_e, E), axis=-1, keepdims=True)
    oh2 = iota_e == i2
    w1 = jnp.sum(jnp.where(oh1, rw, 0.0), axis=-1, keepdims=True)
    w2 = jnp.sum(jnp.where(oh2, rw, 0.0), axis=-1, keepdims=True)
    gates = (oh1 * w1 + oh2 * w2) / (w1 + w2)
    gates_ref[...] = gates.astype(jnp.float32)


HP = H // 2


def _pack16(a16):
    ai = lax.bitcast_convert_type(a16, jnp.int16)
    lo = ai[:, :HP].astype(jnp.int32) & 0xFFFF
    hi = ai[:, HP:].astype(jnp.int32) << 16
    return lo | hi


def _unpack16(p):
    lo = lax.bitcast_convert_type(p << 16, jnp.float32)
    hi = lax.bitcast_convert_type(p & jnp.int32(-65536), jnp.float32)
    return jnp.concatenate([lo, hi], axis=1)


def _make_sc_gather(d, b_rows, dtype, n_chunks=2):
    info = plsc.get_sparse_core_info()
    nw = info.num_cores * info.num_subcores
    b_per_w = b_rows // nw
    chunk = b_per_w // n_chunks
    assert b_per_w % n_chunks == 0 and chunk % 8 == 0
    mesh = plsc.VectorSubcoreMesh(core_axis_name="c", subcore_axis_name="s")

    @functools.partial(
        pl.kernel, mesh=mesh,
        out_type=jax.ShapeDtypeStruct((b_rows, d), dtype),
        scratch_types=[
            pltpu.VMEM((b_per_w,), jnp.int32),
            pltpu.VMEM((chunk, d), dtype),
            pltpu.VMEM((chunk, d), dtype),
            pltpu.SemaphoreType.DMA,
            pltpu.SemaphoreType.DMA,
            pltpu.SemaphoreType.DMA,
            pltpu.SemaphoreType.DMA,
        ],
    )
    def g(table_hbm, idx_hbm, out_hbm, idx_v, rows_a, rows_b, g_a, g_b, o_a, o_b):
        wid = lax.axis_index("s") * info.num_cores + lax.axis_index("c")
        base = wid * b_per_w
        pltpu.sync_copy(idx_hbm.at[pl.ds(base, b_per_w)], idx_v)
        rows = (rows_a, rows_b)
        gsem = (g_a, g_b)
        osem = (o_a, o_b)
        gh = [None, None]
        oh = [None, None]
        gh[0] = pltpu.async_copy(
            table_hbm.at[idx_v.at[pl.ds(0, chunk)]], rows[0], gsem[0])
        for c in range(n_chunks):
            b = c % 2
            nb = (c + 1) % 2
            if c + 1 < n_chunks:
                if oh[nb] is not None:
                    oh[nb].wait()
                gh[nb] = pltpu.async_copy(
                    table_hbm.at[idx_v.at[pl.ds((c + 1) * chunk, chunk)]],
                    rows[nb], gsem[nb])
            gh[b].wait()
            oh[b] = pltpu.async_copy(
                rows[b], out_hbm.at[pl.ds(base + c * chunk, chunk)], osem[b])
        oh[(n_chunks - 2) % 2].wait()
        oh[(n_chunks - 1) % 2].wait()

    return g


def _moe_grouped_body(eot_ref, xs_ref, w1_ref, b1_ref, w2_ref, b2_ref, ys_ref):
    xb = _unpack16(xs_ref[...]).astype(jnp.bfloat16)
    h = jnp.dot(xb, w1_ref[0], preferred_element_type=jnp.float32) + b1_ref[0]
    a = 0.5 * h * (1.0 + lax.erf(h / math.sqrt(2.0)))
    y = jnp.dot(a.astype(jnp.bfloat16), w2_ref[0],
                preferred_element_type=jnp.float32) + b2_ref[0]
    ys_ref[...] = _pack16(y.astype(jnp.bfloat16))


def _combine_body(hidden_ref, gates_ref, y1_ref, y2_ref, out_ref):
    g = gates_ref[...]
    w1 = jnp.max(g, axis=-1, keepdims=True)
    w2 = jnp.sum(g, axis=-1, keepdims=True) - w1
    y1 = _unpack16(y1_ref[...])
    y2 = _unpack16(y2_ref[...])
    out_ref[...] = hidden_ref[...] + w1 * y1 + w2 * y2


def kernel(hidden_states, ln1_g, ln1_b, Wq, bq, Wk, bk, Wv, bv, Wl, bl,
           ln2_g, ln2_b, Wr, br, W1, b1, W2, b2):
    x = hidden_states.reshape(S, H)
    bf = jnp.bfloat16
    Wq16, Wk16, Wv16, Wr16 = (w.astype(bf) for w in (Wq, Wk, Wv, Wr))
    W1_16, W2_16 = W1.astype(bf), W2.astype(bf)

    row = lambda a: a.reshape(1, -1)

    hidden, x2, gates = pl.pallas_call(
        _attn_router_body,
        grid=(NST,),
        in_specs=[
            pl.BlockSpec((TS, H), lambda s: (s, 0)),
            pl.BlockSpec((1, H), lambda s: (0, 0)),
            pl.BlockSpec((1, H), lambda s: (0, 0)),
            pl.BlockSpec((H, H), lambda s: (0, 0)),
            pl.BlockSpec((1, H), lambda s: (0, 0)),
            pl.BlockSpec((H, H), lambda s: (0, 0)),
            pl.BlockSpec((1, H), lambda s: (0, 0)),
            pl.BlockSpec((H, H), lambda s: (0, 0)),
            pl.BlockSpec((1, H), lambda s: (0, 0)),
            pl.BlockSpec((1, H), lambda s: (0, 0)),
            pl.BlockSpec((1, H), lambda s: (0, 0)),
            pl.BlockSpec((H, E), lambda s: (0, 0)),
            pl.BlockSpec((1, E), lambda s: (0, 0)),
        ],
        out_specs=[
            pl.BlockSpec((TS, H), lambda s: (s, 0)),
            pl.BlockSpec((TS, HP), lambda s: (s, 0)),
            pl.BlockSpec((TS, E), lambda s: (s, 0)),
        ],
        out_shape=[
            jax.ShapeDtypeStruct((S, H), jnp.float32),
            jax.ShapeDtypeStruct((S, HP), jnp.int32),
            jax.ShapeDtypeStruct((S, E), jnp.float32),
        ],
    )(x, row(ln1_g), row(ln1_b), Wq16, row(bq), Wk16, row(bk), Wv16, row(bv),
      row(ln2_g), row(ln2_b), Wr16, row(br))

    iota_e = jnp.arange(E, dtype=jnp.int32)[None, :]
    i1 = jnp.argmax(gates, axis=-1).astype(jnp.int32)
    masked = jnp.where(iota_e == i1[:, None], -1.0, gates)
    i2 = jnp.argmax(masked, axis=-1).astype(jnp.int32)
    e_all = jnp.concatenate([i1, i2])
    oh = (e_all[:, None] == iota_e).astype(jnp.int32)
    cum = jnp.cumsum(oh, axis=0)
    rank = jnp.take_along_axis(cum, e_all[:, None], axis=1)[:, 0] - 1
    counts = cum[-1]
    padded = ((counts + TB - 1) // TB) * TB
    ends = jnp.cumsum(padded)
    off = ends - padded
    dest = off[e_all] + rank
    src_tok = jnp.concatenate([jnp.arange(S, dtype=jnp.int32)] * 2)
    src_idx = jnp.zeros((NP,), jnp.int32).at[dest].set(src_tok)
    pos12 = dest
    t_starts = jnp.arange(NT, dtype=jnp.int32) * TB
    eot = jnp.minimum(
        jnp.sum((t_starts[:, None] >= ends[None, :]).astype(jnp.int32), axis=1),
        E - 1).astype(jnp.int32)

    xs = _make_sc_gather(HP, NP, jnp.int32)(x2, src_idx)

    ys = pl.pallas_call(
        _moe_grouped_body,
        grid_spec=pltpu.PrefetchScalarGridSpec(
            num_scalar_prefetch=1,
            grid=(NT,),
            in_specs=[
                pl.BlockSpec((TB, HP), lambda t, eot_ref: (t, 0)),
                pl.BlockSpec((1, H, F), lambda t, eot_ref: (eot_ref[t], 0, 0)),
                pl.BlockSpec((1, 1, F), lambda t, eot_ref: (eot_ref[t], 0, 0)),
                pl.BlockSpec((1, F, H), lambda t, eot_ref: (eot_ref[t], 0, 0)),
                pl.BlockSpec((1, 1, H), lambda t, eot_ref: (eot_ref[t], 0, 0)),
            ],
            out_specs=pl.BlockSpec((TB, HP), lambda t, eot_ref: (t, 0)),
        ),
        out_shape=jax.ShapeDtypeStruct((NP, HP), jnp.int32),
    )(eot, xs, W1_16, b1.reshape(E, 1, F), W2_16, b2.reshape(E, 1, H))

    y12 = _make_sc_gather(HP, K * S, jnp.int32)(ys, pos12)

    out = pl.pallas_call(
        _combine_body,
        grid=(NST,),
        in_specs=[
            pl.BlockSpec((TS, H), lambda s: (s, 0)),
            pl.BlockSpec((TS, E), lambda s: (s, 0)),
            pl.BlockSpec((TS, HP), lambda s: (s, 0)),
            pl.BlockSpec((TS, HP), lambda s: (s + NST, 0)),
        ],
        out_specs=pl.BlockSpec((TS, H), lambda s: (s, 0)),
        out_shape=jax.ShapeDtypeStruct((S, H), jnp.float32),
    )(hidden, gates, y12, y12)

    return out.reshape(B, S, H)
